# baseline (speedup 1.0000x reference)
import functools

import jax
import jax.numpy as jnp
from jax import lax
from jax.experimental import pallas as pl
from jax.experimental.pallas import tpu as pltpu
from jax.experimental.pallas import tpu_sc as plsc

N = 10000
E = 320000
F = 128
T = 4

N_PAD = 10240
CHUNK = 128
NWORKERS = 32
CHUNKS_PER_W = 80
E_PAD = NWORKERS * CHUNKS_PER_W * CHUNK
EDGES_PER_W = CHUNKS_PER_W * CHUNK
IBLK = 16
NIBLK = CHUNKS_PER_W // IBLK
ROWS_PER_TILE = N_PAD // 16

_SELU_SCALE = 1.0507009873554805
_SELU_ALPHA = 1.6732632423543772
_SELU_AS = _SELU_SCALE * _SELU_ALPHA

_BLK = 512
_NBLK = N_PAD // _BLK


def _sc_edge_body(xa_hbm, xbb_hbm, first_hbm, second_hbm, out0, out1,
                  agg_sh, idx_a, idx_b, rows_a, rows_b, zbuf, sem_g, sem_s):
    cid = lax.axis_index("c")
    sid = lax.axis_index("s")
    wid = cid * 16 + sid

    zero = jnp.zeros((16,), jnp.float32)

    @pl.loop(0, 16)
    def _zrow(i):
        for k in range(F // 16):
            zbuf[i, pl.ds(k * 16, 16)] = zero

    zbase = sid * ROWS_PER_TILE

    @pl.loop(0, ROWS_PER_TILE // 16)
    def _zagg(j):
        pltpu.sync_copy(zbuf, agg_sh.at[pl.ds(zbase + j * 16, 16)])

    plsc.subcore_barrier()

    def chunk(g):
        pltpu.async_copy(xa_hbm.at[idx_a.at[g]], rows_a, sem_g)
        pltpu.async_copy(xbb_hbm.at[idx_b.at[g]], rows_b, sem_g)
        pltpu.make_async_copy(xa_hbm.at[idx_a.at[g]], rows_a, sem_g).wait()
        pltpu.make_async_copy(xbb_hbm.at[idx_b.at[g]], rows_b, sem_g).wait()

        @plsc.parallel_loop(0, CHUNK, unroll=4)
        def _row(i):
            for k in range(F // 16):
                sl = pl.ds(k * 16, 16)
                x = rows_a[i, sl] + rows_b[i, sl]
                e = jnp.exp(jnp.minimum(x, 0.0))
                rows_b[i, sl] = (_SELU_SCALE * jnp.maximum(x, 0.0)
                                 + (_SELU_AS * e - _SELU_AS))

        pltpu.async_copy(rows_b, agg_sh.at[idx_b.at[g]], sem_s, add=True)
        pltpu.make_async_copy(rows_b, agg_sh.at[idx_b.at[g]], sem_s).wait()

    def block(q):
        pltpu.sync_copy(first_hbm.at[wid, pl.ds(q * IBLK, IBLK)], idx_a)
        pltpu.sync_copy(second_hbm.at[wid, pl.ds(q * IBLK, IBLK)], idx_b)

        @pl.loop(0, IBLK)
        def _c(g):
            chunk(g)

    @pl.loop(0, NIBLK)
    def _blk(q):
        block(q)

    plsc.subcore_barrier()

    @pl.loop(0, ROWS_PER_TILE // 32)
    def _wout(j):
        roff = sid * ROWS_PER_TILE + j * 32
        sl = pl.ds(roff, 32)

        @pl.when(cid == 0)
        def _():
            pltpu.sync_copy(agg_sh.at[sl], out0.at[sl])

        @pl.when(cid == 1)
        def _():
            pltpu.sync_copy(agg_sh.at[sl], out1.at[sl])


@functools.cache
def _sc_edge():
    return pl.kernel(
        _sc_edge_body,
        out_type=[jax.ShapeDtypeStruct((N_PAD, F), jnp.float32),
                  jax.ShapeDtypeStruct((N_PAD, F), jnp.float32)],
        mesh=plsc.VectorSubcoreMesh(core_axis_name="c", subcore_axis_name="s",
                                    num_cores=2, num_subcores=16),
        scratch_types=[
            pltpu.VMEM_SHARED((N_PAD, F), jnp.float32),
            pltpu.VMEM((IBLK, CHUNK), jnp.int32),
            pltpu.VMEM((IBLK, CHUNK), jnp.int32),
            pltpu.VMEM((CHUNK, F), jnp.float32),
            pltpu.VMEM((CHUNK, F), jnp.float32),
            pltpu.VMEM((16, F), jnp.float32),
            pltpu.SemaphoreType.DMA,
            pltpu.SemaphoreType.DMA,
        ],
    )


def _dot_t(x, w):
    return lax.dot_general(x, w, (((1,), (1,)), ((), ())),
                           preferred_element_type=jnp.float32)


def _msg_tables_body(state_ref, msg_w_ref, msg_b_ref, xa_ref, xbb_ref):
    s = state_ref[...]
    w = msg_w_ref[...]
    xa_ref[...] = _dot_t(s, w[:, :F])
    xbb_ref[...] = _dot_t(s, w[:, F:]) + msg_b_ref[...]


def _msg_tables(state_pad, msg_w, msg_b2d):
    blk = lambda i: (i, 0)
    whole = lambda i: (0, 0)
    return pl.pallas_call(
        _msg_tables_body,
        grid=(_NBLK,),
        in_specs=[pl.BlockSpec((_BLK, F), blk),
                  pl.BlockSpec((F, 2 * F), whole),
                  pl.BlockSpec((1, F), whole)],
        out_specs=[pl.BlockSpec((_BLK, F), blk),
                   pl.BlockSpec((_BLK, F), blk)],
        out_shape=[jax.ShapeDtypeStruct((N_PAD, F), jnp.float32),
                   jax.ShapeDtypeStruct((N_PAD, F), jnp.float32)],
    )(state_pad, msg_w, msg_b2d)


def _gru_body(state_ref, agg0_ref, agg1_ref, wih_ref, whh_ref, bih_ref,
              bhh_ref, msg_w_ref, msg_b_ref, ns_ref, xa_ref, xbb_ref,
              *, tables):
    s = state_ref[...]
    a = agg0_ref[...] + agg1_ref[...]
    gi = _dot_t(a, wih_ref[...]) + bih_ref[...]
    gh = _dot_t(s, whh_ref[...]) + bhh_ref[...]
    r = jax.nn.sigmoid(gi[:, :F] + gh[:, :F])
    z = jax.nn.sigmoid(gi[:, F:2 * F] + gh[:, F:2 * F])
    n = jnp.tanh(gi[:, 2 * F:] + r * gh[:, 2 * F:])
    ns = (1.0 - z) * n + z * s
    ns_ref[...] = ns
    if tables:
        w = msg_w_ref[...]
        xa_ref[...] = _dot_t(ns, w[:, :F])
        xbb_ref[...] = _dot_t(ns, w[:, F:]) + msg_b_ref[...]


def _gru_step(state_pad, agg0, agg1, wih, whh, bih2d, bhh2d, msg_w, msg_b2d,
              tables):
    blk = lambda i: (i, 0)
    whole = lambda i: (0, 0)
    n_out = 3 if tables else 1

    def body(*refs):
        if tables:
            _gru_body(*refs, tables=True)
        else:
            _gru_body(*refs, None, None, tables=False)

    out = pl.pallas_call(
        body,
        grid=(_NBLK,),
        in_specs=[pl.BlockSpec((_BLK, F), blk),
                  pl.BlockSpec((_BLK, F), blk),
                  pl.BlockSpec((_BLK, F), blk),
                  pl.BlockSpec((3 * F, F), whole),
                  pl.BlockSpec((3 * F, F), whole),
                  pl.BlockSpec((1, 3 * F), whole),
                  pl.BlockSpec((1, 3 * F), whole),
                  pl.BlockSpec((F, 2 * F), whole),
                  pl.BlockSpec((1, F), whole)],
        out_specs=[pl.BlockSpec((_BLK, F), blk)] * n_out,
        out_shape=[jax.ShapeDtypeStruct((N_PAD, F), jnp.float32)] * n_out,
    )(state_pad, agg0, agg1, wih, whh, bih2d, bhh2d, msg_w, msg_b2d)
    if tables:
        return out[0], out[1], out[2]
    return out[0], None, None


def _selu(x):
    return jnp.where(x > 0.0, _SELU_SCALE * x,
                     _SELU_AS * jnp.exp(x) - _SELU_AS)


def _readout_body(state_ref, r1w_ref, r1b_ref, r2w_ref, r2b_ref, outw_ref,
                  outb_ref, out_ref):
    s = state_ref[...]
    rows = lax.broadcasted_iota(jnp.int32, (N_PAD, F), 0)
    feat = jnp.sum(jnp.where(rows < N, s, 0.0), axis=0, keepdims=True)
    h = _selu(_dot_t(feat, r1w_ref[...]) + r1b_ref[...])
    h = _selu(_dot_t(h, r2w_ref[...]) + r2b_ref[...])
    out_ref[...] = jnp.sum(h * outw_ref[...]) + outb_ref[...]


def _readout(state_pad, r1w, r1b2d, r2w, r2b2d, outw, outb2d):
    return pl.pallas_call(
        _readout_body,
        out_shape=jax.ShapeDtypeStruct((1, F), jnp.float32),
    )(state_pad, r1w, r1b2d, r2w, r2b2d, outw, outb2d)


def kernel(link_state, first, second, state_dim, msg_W, msg_b,
           gru_W_ih, gru_W_hh, gru_b_ih, gru_b_hh,
           r1_W, r1_b, r2_W, r2_b, out_W, out_b):
    first_p = jnp.pad(first, (0, E_PAD - E), constant_values=N).reshape(
        NWORKERS, CHUNKS_PER_W, CHUNK)
    second_p = jnp.pad(second, (0, E_PAD - E), constant_values=N).reshape(
        NWORKERS, CHUNKS_PER_W, CHUNK)
    state_pad = jnp.pad(link_state, ((0, N_PAD - N), (0, 0)))

    msg_b2d = msg_b.reshape(1, F)
    bih2d = gru_b_ih.reshape(1, 3 * F)
    bhh2d = gru_b_hh.reshape(1, 3 * F)

    xa, xbb = _msg_tables(state_pad, msg_W, msg_b2d)
    for t in range(T):
        agg0, agg1 = _sc_edge()(xa, xbb, first_p, second_p)
        state_pad, xa, xbb = _gru_step(
            state_pad, agg0, agg1, gru_W_ih, gru_W_hh, bih2d, bhh2d,
            msg_W, msg_b2d, tables=(t < T - 1))

    out = _readout(state_pad, r1_W, r1_b.reshape(1, F), r2_W,
                   r2_b.reshape(1, F), out_W,
                   jnp.broadcast_to(out_b.reshape(1, 1), (1, F)))
    return out[0, :1]


# speedup vs baseline: 1.3498x; 1.3498x over previous
import functools

import jax
import jax.numpy as jnp
from jax import lax
from jax.experimental import pallas as pl
from jax.experimental.pallas import tpu as pltpu
from jax.experimental.pallas import tpu_sc as plsc

N = 10000
E = 320000
F = 128
T = 4

N_PAD = 10240
CHUNK = 128
NWORKERS = 32
CHUNKS_PER_W = -(-E // (NWORKERS * CHUNK))
E_PAD = NWORKERS * CHUNKS_PER_W * CHUNK
EDGES_PER_W = CHUNKS_PER_W * CHUNK
ROWS_PER_TILE = N_PAD // 16

_SELU_SCALE = 1.0507009873554805
_SELU_ALPHA = 1.6732632423543772
_SELU_AS = _SELU_SCALE * _SELU_ALPHA

_BLK = 512
_NBLK = N_PAD // _BLK


def _sc_edge_body(xa_hbm, xbb_hbm, first_hbm, second_hbm, out0, out1,
                  agg_sh, idx_a, idx_b, rows_a, rows_b, zbuf, sem_a, sem_b):
    cid = lax.axis_index("c")
    sid = lax.axis_index("s")

    zero = jnp.zeros((16,), jnp.float32)

    @pl.loop(0, 16)
    def _zrow(i):
        for k in range(F // 16):
            zbuf[i, pl.ds(k * 16, 16)] = zero

    zbase = sid * ROWS_PER_TILE

    @pl.loop(0, ROWS_PER_TILE // 16)
    def _zagg(j):
        pltpu.sync_copy(zbuf, agg_sh.at[pl.ds(zbase + j * 16, 16)])

    plsc.subcore_barrier()

    wid = cid * 16 + sid
    ebase = wid * EDGES_PER_W

    @pl.loop(0, CHUNKS_PER_W)
    def _chunk(g):
        eoff = ebase + g * CHUNK
        pltpu.sync_copy(first_hbm.at[pl.ds(eoff, CHUNK)], idx_a)
        pltpu.sync_copy(second_hbm.at[pl.ds(eoff, CHUNK)], idx_b)
        cp_a = pltpu.async_copy(xa_hbm.at[idx_a], rows_a, sem_a)
        cp_b = pltpu.async_copy(xbb_hbm.at[idx_b], rows_b, sem_b)
        cp_a.wait()
        cp_b.wait()

        @pl.loop(0, CHUNK)
        def _row(i):
            for k in range(F // 16):
                sl = pl.ds(k * 16, 16)
                x = rows_a[i, sl] + rows_b[i, sl]
                neg = _SELU_AS * jnp.exp(x) - _SELU_AS
                rows_a[i, sl] = jnp.where(x > 0.0, _SELU_SCALE * x, neg)

        pltpu.sync_copy(rows_a, agg_sh.at[idx_b], add=True)

    plsc.subcore_barrier()

    @pl.loop(0, ROWS_PER_TILE // CHUNK)
    def _wout(j):
        roff = sid * ROWS_PER_TILE + j * CHUNK
        pltpu.sync_copy(agg_sh.at[pl.ds(roff, CHUNK)], rows_a)

        @pl.when(cid == 0)
        def _():
            pltpu.sync_copy(rows_a, out0.at[pl.ds(roff, CHUNK)])

        @pl.when(cid == 1)
        def _():
            pltpu.sync_copy(rows_a, out1.at[pl.ds(roff, CHUNK)])


@functools.cache
def _sc_edge():
    return pl.kernel(
        _sc_edge_body,
        out_type=[jax.ShapeDtypeStruct((N_PAD, F), jnp.float32),
                  jax.ShapeDtypeStruct((N_PAD, F), jnp.float32)],
        mesh=plsc.VectorSubcoreMesh(core_axis_name="c", subcore_axis_name="s",
                                    num_cores=2, num_subcores=16),
        scratch_types=[
            pltpu.VMEM_SHARED((N_PAD, F), jnp.float32),
            pltpu.VMEM((CHUNK,), jnp.int32),
            pltpu.VMEM((CHUNK,), jnp.int32),
            pltpu.VMEM((CHUNK, F), jnp.float32),
            pltpu.VMEM((CHUNK, F), jnp.float32),
            pltpu.VMEM((16, F), jnp.float32),
            pltpu.SemaphoreType.DMA,
            pltpu.SemaphoreType.DMA,
        ],
    )


def _dot_t(x, w):
    return lax.dot_general(x, w, (((1,), (1,)), ((), ())),
                           preferred_element_type=jnp.float32)


def _msg_tables_body(state_ref, msg_w_ref, msg_b_ref, xa_ref, xbb_ref):
    s = state_ref[...]
    w = msg_w_ref[...]
    xa_ref[...] = _dot_t(s, w[:, :F])
    xbb_ref[...] = _dot_t(s, w[:, F:]) + msg_b_ref[...]


def _msg_tables(state_pad, msg_w, msg_b2d):
    blk = lambda i: (i, 0)
    whole = lambda i: (0, 0)
    return pl.pallas_call(
        _msg_tables_body,
        grid=(_NBLK,),
        in_specs=[pl.BlockSpec((_BLK, F), blk),
                  pl.BlockSpec((F, 2 * F), whole),
                  pl.BlockSpec((1, F), whole)],
        out_specs=[pl.BlockSpec((_BLK, F), blk),
                   pl.BlockSpec((_BLK, F), blk)],
        out_shape=[jax.ShapeDtypeStruct((N_PAD, F), jnp.float32),
                   jax.ShapeDtypeStruct((N_PAD, F), jnp.float32)],
    )(state_pad, msg_w, msg_b2d)


def _gru_body(state_ref, agg0_ref, agg1_ref, wih_ref, whh_ref, bih_ref,
              bhh_ref, msg_w_ref, msg_b_ref, ns_ref, xa_ref, xbb_ref,
              *, tables):
    s = state_ref[...]
    a = agg0_ref[...] + agg1_ref[...]
    gi = _dot_t(a, wih_ref[...]) + bih_ref[...]
    gh = _dot_t(s, whh_ref[...]) + bhh_ref[...]
    r = jax.nn.sigmoid(gi[:, :F] + gh[:, :F])
    z = jax.nn.sigmoid(gi[:, F:2 * F] + gh[:, F:2 * F])
    n = jnp.tanh(gi[:, 2 * F:] + r * gh[:, 2 * F:])
    ns = (1.0 - z) * n + z * s
    ns_ref[...] = ns
    if tables:
        w = msg_w_ref[...]
        xa_ref[...] = _dot_t(ns, w[:, :F])
        xbb_ref[...] = _dot_t(ns, w[:, F:]) + msg_b_ref[...]


def _gru_step(state_pad, agg0, agg1, wih, whh, bih2d, bhh2d, msg_w, msg_b2d,
              tables):
    blk = lambda i: (i, 0)
    whole = lambda i: (0, 0)
    n_out = 3 if tables else 1

    def body(*refs):
        if tables:
            _gru_body(*refs, tables=True)
        else:
            _gru_body(*refs, None, None, tables=False)

    out = pl.pallas_call(
        body,
        grid=(_NBLK,),
        in_specs=[pl.BlockSpec((_BLK, F), blk),
                  pl.BlockSpec((_BLK, F), blk),
                  pl.BlockSpec((_BLK, F), blk),
                  pl.BlockSpec((3 * F, F), whole),
                  pl.BlockSpec((3 * F, F), whole),
                  pl.BlockSpec((1, 3 * F), whole),
                  pl.BlockSpec((1, 3 * F), whole),
                  pl.BlockSpec((F, 2 * F), whole),
                  pl.BlockSpec((1, F), whole)],
        out_specs=[pl.BlockSpec((_BLK, F), blk)] * n_out,
        out_shape=[jax.ShapeDtypeStruct((N_PAD, F), jnp.float32)] * n_out,
    )(state_pad, agg0, agg1, wih, whh, bih2d, bhh2d, msg_w, msg_b2d)
    if tables:
        return out[0], out[1], out[2]
    return out[0], None, None


def _selu(x):
    return jnp.where(x > 0.0, _SELU_SCALE * x,
                     _SELU_AS * jnp.exp(x) - _SELU_AS)


def _readout_body(state_ref, r1w_ref, r1b_ref, r2w_ref, r2b_ref, outw_ref,
                  outb_ref, out_ref):
    s = state_ref[...]
    rows = lax.broadcasted_iota(jnp.int32, (N_PAD, F), 0)
    feat = jnp.sum(jnp.where(rows < N, s, 0.0), axis=0, keepdims=True)
    h = _selu(_dot_t(feat, r1w_ref[...]) + r1b_ref[...])
    h = _selu(_dot_t(h, r2w_ref[...]) + r2b_ref[...])
    out_ref[...] = jnp.sum(h * outw_ref[...]) + outb_ref[...]


def _readout(state_pad, r1w, r1b2d, r2w, r2b2d, outw, outb2d):
    return pl.pallas_call(
        _readout_body,
        out_shape=jax.ShapeDtypeStruct((1, F), jnp.float32),
    )(state_pad, r1w, r1b2d, r2w, r2b2d, outw, outb2d)


def kernel(link_state, first, second, state_dim, msg_W, msg_b,
           gru_W_ih, gru_W_hh, gru_b_ih, gru_b_hh,
           r1_W, r1_b, r2_W, r2_b, out_W, out_b):
    first_p = jnp.pad(first, (0, E_PAD - E), constant_values=N)
    second_p = jnp.pad(second, (0, E_PAD - E), constant_values=N)
    state_pad = jnp.pad(link_state, ((0, N_PAD - N), (0, 0)))

    msg_b2d = msg_b.reshape(1, F)
    bih2d = gru_b_ih.reshape(1, 3 * F)
    bhh2d = gru_b_hh.reshape(1, 3 * F)

    xa, xbb = _msg_tables(state_pad, msg_W, msg_b2d)
    for t in range(T):
        agg0, agg1 = _sc_edge()(xa, xbb, first_p, second_p)
        state_pad, xa, xbb = _gru_step(
            state_pad, agg0, agg1, gru_W_ih, gru_W_hh, bih2d, bhh2d,
            msg_W, msg_b2d, tables=(t < T - 1))

    out = _readout(state_pad, r1_W, r1_b.reshape(1, F), r2_W,
                   r2_b.reshape(1, F), out_W,
                   jnp.broadcast_to(out_b.reshape(1, 1), (1, F)))
    return out[0, :1]
